# baseline (speedup 1.0000x reference)
import jax
import jax.numpy as jnp
from jax.experimental import pallas as pl
from jax.experimental.pallas import tpu as pltpu


def _fill_and_copy_kernel(logps_ref, out_ref, scratch_ref, sem):
    scratch_ref[...] = jnp.broadcast_to(
        logps_ref[...][:, None, :], scratch_ref.shape
    )
    n = out_ref.shape[0]
    for i in range(n):
        pltpu.make_async_copy(scratch_ref, out_ref.at[pl.ds(i, 1)], sem).start()
    for i in range(n):
        pltpu.make_async_copy(scratch_ref, out_ref.at[pl.ds(i, 1)], sem).wait()


def kernel(hist, logps):
    S, B = hist.shape
    V = logps.shape[0]
    logps2d = logps.reshape(1, V)

    out = pl.pallas_call(
        _fill_and_copy_kernel,
        in_specs=[pl.BlockSpec((1, V), lambda: (0, 0))],
        out_specs=pl.BlockSpec(memory_space=pltpu.MemorySpace.HBM),
        out_shape=jax.ShapeDtypeStruct((S + 1, B, V), jnp.float32),
        scratch_shapes=[
            pltpu.VMEM((1, B, V), jnp.float32),
            pltpu.SemaphoreType.DMA,
        ],
    )(logps2d)
    return out


# speedup vs baseline: 1.0167x; 1.0167x over previous
import jax
import jax.numpy as jnp
from jax.experimental import pallas as pl
from jax.experimental.pallas import tpu as pltpu


def _fill_and_copy_kernel(logps_ref, out_ref, scratch_ref, sems):
    scratch_ref[...] = jnp.broadcast_to(
        logps_ref[...][:, None, :], scratch_ref.shape
    )
    n = out_ref.shape[0]
    for i in range(n):
        pltpu.make_async_copy(
            scratch_ref, out_ref.at[pl.ds(i, 1)], sems.at[i]
        ).start()
    for i in range(n):
        pltpu.make_async_copy(
            scratch_ref, out_ref.at[pl.ds(i, 1)], sems.at[i]
        ).wait()


def kernel(hist, logps):
    S, B = hist.shape
    V = logps.shape[0]
    logps2d = logps.reshape(1, V)

    out = pl.pallas_call(
        _fill_and_copy_kernel,
        in_specs=[pl.BlockSpec((1, V), lambda: (0, 0))],
        out_specs=pl.BlockSpec(memory_space=pltpu.MemorySpace.HBM),
        out_shape=jax.ShapeDtypeStruct((S + 1, B, V), jnp.float32),
        scratch_shapes=[
            pltpu.VMEM((1, B, V), jnp.float32),
            pltpu.SemaphoreType.DMA((S + 1,)),
        ],
    )(logps2d)
    return out
